# baseline (speedup 1.0000x reference)
import functools

import jax
import jax.numpy as jnp
from jax import lax
from jax.experimental import pallas as pl
from jax.experimental.pallas import tpu as pltpu
from jax.experimental.pallas import tpu_sc as plsc

B, L = 1024, 50
E, H = 64, 128
ROWS = B * L

NSC = 5

OH_N = (20, 100, 100, 10, 10, 24, 7, 2)
OH_PAD = (24, 104, 104, 16, 16, 24, 8, 8)
OH_TOT = sum(OH_PAD)
NOH = len(OH_N)

NC, NS = 2, 16
NW = NC * NS
CHUNK = 128
NCHK = ROWS // CHUNK
ITEMS = 13
NBUF = 5
NBATCH = 3

BR = 800
NBLK = ROWS // BR


def _gather_body(*refs):
    idxs = refs[:NSC]
    tables = refs[NSC:2 * NSC]
    out_hbm = refs[2 * NSC]
    idx_v, rows_v, sem_i, sem_g, sem_w = refs[2 * NSC + 1:]
    wid = lax.axis_index("s") * NC + lax.axis_index("c")

    def chunk_of(j):
        c = jnp.minimum(wid + NW * j, NCHK - 1)
        return pl.multiple_of(c * CHUNK, CHUNK)

    fetches = []
    for f in range(NSC):
        for j in range(ITEMS):
            fetches.append(pltpu.async_copy(
                idxs[f].at[pl.ds(chunk_of(j), CHUNK)], idx_v.at[f, j],
                sem_i))
    for d in fetches:
        d.wait()

    for f in range(NSC):
        @pl.loop(0, NBATCH)
        def _(m, f=f):
            for b in range(NBUF):
                j = m * NBUF + b
                @pl.when(j < ITEMS)
                def _():
                    pltpu.async_copy(
                        tables[f].at[idx_v.at[f, j]], rows_v.at[b], sem_g)
            for b in range(NBUF):
                j = m * NBUF + b
                @pl.when(j < ITEMS)
                def _():
                    pltpu.make_async_copy(
                        tables[f].at[idx_v.at[f, j]], rows_v.at[b],
                        sem_g).wait()
                    pltpu.async_copy(
                        rows_v.at[b], out_hbm.at[f].at[pl.ds(chunk_of(j),
                                                             CHUNK)], sem_w)
            for b in range(NBUF):
                j = m * NBUF + b
                @pl.when(j < ITEMS)
                def _():
                    pltpu.make_async_copy(
                        rows_v.at[b], out_hbm.at[f].at[pl.ds(chunk_of(j),
                                                             CHUNK)],
                        sem_w).wait()


_sc_gather = functools.partial(
    pl.kernel,
    mesh=plsc.VectorSubcoreMesh(core_axis_name="c", subcore_axis_name="s"),
    out_type=jax.ShapeDtypeStruct((NSC, ROWS, H), jnp.float32),
    scratch_types=[
        pltpu.VMEM((NSC, ITEMS, CHUNK), jnp.int32),
        pltpu.VMEM((NBUF, CHUNK, H), jnp.float32),
        pltpu.SemaphoreType.DMA,
        pltpu.SemaphoreType.DMA,
        pltpu.SemaphoreType.DMA,
    ],
)(_gather_body)


def _prep_body(wi_ref, wl_ref, bi_ref, bl_ref, pos_ref, price_ref, nch_ref,
               ela_ref, evt_ref, pa_ref, hour_ref, wday_ref, wend_ref,
               m_ref, p_ref, md_ref, mi_ref, pe_ref):
    wl0 = wl_ref[:, :H]
    c = jnp.dot(wl0, wi_ref[...], preferred_element_type=jnp.float32)
    m_ref[...] = jnp.concatenate(
        [c[:, 0:64], c[:, 192:256], c[:, 256:320], c[:, 320:384],
         wl_ref[:, 320:384]], axis=1)
    p_ref[...] = jnp.zeros((OH_TOT, H), jnp.float32)
    def proj(t_ref, rhs):
        return lax.dot_general(t_ref[...], rhs, (((1,), (1,)), ((), ())),
                               preferred_element_type=jnp.float32)
    p_ref[0:20] = proj(price_ref, c[:, 64:128])
    p_ref[24:124] = proj(nch_ref, c[:, 128:192])
    p_ref[128:228] = proj(ela_ref, wl_ref[:, 128:192])
    p_ref[232:242] = proj(evt_ref, wl_ref[:, 192:256])
    p_ref[248:258] = proj(pa_ref, wl_ref[:, 256:320])
    p_ref[264:288] = proj(hour_ref, wl_ref[:, 384:448])
    p_ref[288:295] = proj(wday_ref, wl_ref[:, 448:512])
    p_ref[296:298] = proj(wend_ref, wl_ref[:, 512:576])
    md_ref[...] = c[:, 384:434]
    mi_ref[...] = c[:, 434:484]
    beff = lax.dot_general(bi_ref[...], wl0, (((1,), (1,)), ((), ())),
                           preferred_element_type=jnp.float32)
    pe_ref[...] = pos_ref[...] + beff + bl_ref[...]


def _prep(w_item, w_lin, b_item, b_lin, pos_table, small_tables):
    return pl.pallas_call(
        _prep_body,
        out_shape=(
            jax.ShapeDtypeStruct((H, NSC * E), jnp.float32),
            jax.ShapeDtypeStruct((OH_TOT, H), jnp.float32),
            jax.ShapeDtypeStruct((H, 50), jnp.float32),
            jax.ShapeDtypeStruct((H, 50), jnp.float32),
            jax.ShapeDtypeStruct((L, H), jnp.float32),
        ),
    )(w_item, w_lin, b_item.reshape(1, H), b_lin.reshape(1, H), pos_table,
      *small_tables)


def _main_body(*refs):
    gs = refs[:NSC]
    (par_ref, sidx_ref, desc_ref, img_ref, m_ref, p_ref, md_ref, mi_ref,
     pe_ref, g_ref, b_ref, o_ref) = refs[NSC:]
    bf = jnp.bfloat16
    parts = []
    for f in range(NSC):
        g = gs[f][0]
        pf = par_ref[f, 0].astype(jnp.float32)
        parts.append(g[:, :E] + (g[:, E:] - g[:, :E]) * pf)
    xg = jnp.concatenate(parts, axis=-1)
    y = lax.dot_general(xg.astype(bf), m_ref[...].astype(bf),
                        (((1,), (1,)), ((), ())),
                        preferred_element_type=jnp.float32)
    oh_parts = []
    for f in range(NOH):
        iota = lax.broadcasted_iota(jnp.int32, (OH_PAD[f], BR), 0)
        oh_parts.append((iota == sidx_ref[f, 0]).astype(bf))
    oht = jnp.concatenate(oh_parts, axis=0)
    y += lax.dot_general(oht, p_ref[...].astype(bf),
                         (((0,), (0,)), ((), ())),
                         preferred_element_type=jnp.float32)
    y += lax.dot_general(desc_ref[...].astype(bf), md_ref[...].astype(bf),
                         (((1,), (1,)), ((), ())),
                         preferred_element_type=jnp.float32)
    y += lax.dot_general(img_ref[...].astype(bf), mi_ref[...].astype(bf),
                         (((1,), (1,)), ((), ())),
                         preferred_element_type=jnp.float32)
    y = (y.reshape(BR // L, L, H) + pe_ref[...][None]).reshape(BR, H)
    mean = jnp.mean(y, axis=-1, keepdims=True)
    yc = y - mean
    var = jnp.mean(yc * yc, axis=-1, keepdims=True)
    o_ref[...] = yc * lax.rsqrt(var + 1e-12) * g_ref[...] + b_ref[...]


def _field_spec(f):
    return pl.BlockSpec((1, BR, H), lambda i, f=f: (f, i, 0))


def _main(gx, par, sidx, desc, img, m3, p, md, mi, pe, gamma, beta):
    return pl.pallas_call(
        _main_body,
        grid=(NBLK,),
        in_specs=[_field_spec(f) for f in range(NSC)] + [
            pl.BlockSpec((NSC, 1, BR, 1), lambda i: (0, i, 0, 0)),
            pl.BlockSpec((NOH, 1, 1, BR), lambda i: (0, i, 0, 0)),
            pl.BlockSpec((BR, 50), lambda i: (i, 0)),
            pl.BlockSpec((BR, 50), lambda i: (i, 0)),
            pl.BlockSpec((H, NSC * E), lambda i: (0, 0)),
            pl.BlockSpec((OH_TOT, H), lambda i: (0, 0)),
            pl.BlockSpec((H, 50), lambda i: (0, 0)),
            pl.BlockSpec((H, 50), lambda i: (0, 0)),
            pl.BlockSpec((L, H), lambda i: (0, 0)),
            pl.BlockSpec((1, H), lambda i: (0, 0)),
            pl.BlockSpec((1, H), lambda i: (0, 0)),
        ],
        out_specs=pl.BlockSpec((BR, H), lambda i: (i, 0)),
        out_shape=jax.ShapeDtypeStruct((ROWS, H), jnp.float32),
        compiler_params=pltpu.CompilerParams(
            dimension_semantics=("parallel",)),
    )(*([gx] * NSC), par, sidx, desc, img, m3, p, md, mi, pe, gamma, beta)


def kernel(input_ids, elapsed_time, event_type, product_action, hashed_url,
           price_bucket, number_of_category_hash, category_hash_first_level,
           category_hash_second_level, category_hash_third_level,
           description_vector, image_vector, hour, weekday, weekend,
           id_table, elapsed_table, event_table, pa_table, url_table,
           price_table, nch_table, c1_table, c2_table, c3_table, hour_table,
           weekday_table, weekend_table, pos_table, W_item, b_item, W_lin,
           b_lin, gamma, beta):
    pair = lambda t: t.reshape(t.shape[0] // 2, H)
    sc_idx = [input_ids.reshape(ROWS),
              category_hash_first_level.reshape(ROWS),
              category_hash_second_level.reshape(ROWS),
              category_hash_third_level.reshape(ROWS),
              hashed_url.reshape(ROWS)]
    gx = _sc_gather(
        *[ix >> 1 for ix in sc_idx],
        pair(id_table), pair(c1_table), pair(c2_table), pair(c3_table),
        pair(url_table))
    par = jnp.stack([ix & 1 for ix in sc_idx]).reshape(NSC, NBLK, BR, 1)
    sidx = jnp.stack([
        price_bucket.reshape(ROWS), number_of_category_hash.reshape(ROWS),
        elapsed_time.reshape(ROWS), event_type.reshape(ROWS),
        product_action.reshape(ROWS), hour.reshape(ROWS),
        weekday.reshape(ROWS), weekend.reshape(ROWS)])
    sidx = sidx.reshape(NOH, NBLK, 1, BR)
    m3, p, md, mi, pe = _prep(
        W_item, W_lin, b_item, b_lin, pos_table,
        (price_table, nch_table, elapsed_table, event_table, pa_table,
         hour_table, weekday_table, weekend_table))
    out = _main(gx, par, sidx, description_vector.reshape(ROWS, 50),
                image_vector.reshape(ROWS, 50), m3, p, md, mi, pe,
                gamma.reshape(1, H), beta.reshape(1, H))
    return out.reshape(B, L, H)


# speedup vs baseline: 1.3286x; 1.3286x over previous
import functools

import jax
import jax.numpy as jnp
from jax import lax
from jax.experimental import pallas as pl
from jax.experimental.pallas import tpu as pltpu
from jax.experimental.pallas import tpu_sc as plsc

B, L = 1024, 50
E, H = 64, 128
ROWS = B * L

NSC = 5

OH_N = (20, 100, 100, 10, 10, 24, 7, 2)
OH_PAD = (24, 104, 104, 16, 16, 24, 8, 8)
OH_TOT = sum(OH_PAD)
NOH = len(OH_N)

NC, NS = 2, 16
NW = NC * NS
CHUNK = 128
NCHK = ROWS // CHUNK
ITEMS = 13
NBUF = 5
NBATCH = 3

BR = 800
NBLK = ROWS // BR


def _gather_body(*refs):
    idxs = refs[:NSC]
    tables = refs[NSC:2 * NSC]
    out_hbm = refs[2 * NSC]
    idx_v, rows_v, sem_i, sem_g, sem_w = refs[2 * NSC + 1:]
    wid = lax.axis_index("s") * NC + lax.axis_index("c")

    def chunk_of(j):
        c = jnp.minimum(wid + NW * j, NCHK - 1)
        return pl.multiple_of(c * CHUNK, CHUNK)

    fetches = []
    for f in range(NSC):
        for j in range(ITEMS):
            fetches.append(pltpu.async_copy(
                idxs[f].at[pl.ds(chunk_of(j), CHUNK)], idx_v.at[f, j],
                sem_i))
    for d in fetches:
        d.wait()

    for f in range(NSC):
        @pl.loop(0, NBATCH)
        def _(m, f=f):
            for b in range(NBUF):
                j = m * NBUF + b
                @pl.when(j < ITEMS)
                def _():
                    pltpu.async_copy(
                        tables[f].at[idx_v.at[f, j]], rows_v.at[b], sem_g)
            for b in range(NBUF):
                j = m * NBUF + b
                @pl.when(j < ITEMS)
                def _():
                    pltpu.make_async_copy(
                        tables[f].at[idx_v.at[f, j]], rows_v.at[b],
                        sem_g).wait()
                    pltpu.async_copy(
                        rows_v.at[b], out_hbm.at[f].at[pl.ds(chunk_of(j),
                                                             CHUNK)], sem_w)
            for b in range(NBUF):
                j = m * NBUF + b
                @pl.when(j < ITEMS)
                def _():
                    pltpu.make_async_copy(
                        rows_v.at[b], out_hbm.at[f].at[pl.ds(chunk_of(j),
                                                             CHUNK)],
                        sem_w).wait()


_sc_gather = functools.partial(
    pl.kernel,
    mesh=plsc.VectorSubcoreMesh(core_axis_name="c", subcore_axis_name="s"),
    out_type=jax.ShapeDtypeStruct((NSC, ROWS, H), jnp.float32),
    scratch_types=[
        pltpu.VMEM((NSC, ITEMS, CHUNK), jnp.int32),
        pltpu.VMEM((NBUF, CHUNK, H), jnp.float32),
        pltpu.SemaphoreType.DMA,
        pltpu.SemaphoreType.DMA,
        pltpu.SemaphoreType.DMA,
    ],
)(_gather_body)


def _prep_body(wi_ref, wl_ref, bi_ref, bl_ref, pos_ref, price_ref, nch_ref,
               ela_ref, evt_ref, pa_ref, hour_ref, wday_ref, wend_ref,
               m_ref, p_ref, md_ref, mi_ref, pe_ref):
    wl0 = wl_ref[:, :H]
    c = jnp.dot(wl0, wi_ref[...], preferred_element_type=jnp.float32)
    z = jnp.zeros((H, E), jnp.float32)
    m_ref[...] = jnp.concatenate(
        [c[:, 0:64], z, c[:, 192:256], z, c[:, 256:320], z,
         c[:, 320:384], z, wl_ref[:, 320:384], z], axis=1)
    p_ref[...] = jnp.zeros((OH_TOT, H), jnp.float32)
    def proj(t_ref, rhs):
        return lax.dot_general(t_ref[...], rhs, (((1,), (1,)), ((), ())),
                               preferred_element_type=jnp.float32)
    p_ref[0:20] = proj(price_ref, c[:, 64:128])
    p_ref[24:124] = proj(nch_ref, c[:, 128:192])
    p_ref[128:228] = proj(ela_ref, wl_ref[:, 128:192])
    p_ref[232:242] = proj(evt_ref, wl_ref[:, 192:256])
    p_ref[248:258] = proj(pa_ref, wl_ref[:, 256:320])
    p_ref[264:288] = proj(hour_ref, wl_ref[:, 384:448])
    p_ref[288:295] = proj(wday_ref, wl_ref[:, 448:512])
    p_ref[296:298] = proj(wend_ref, wl_ref[:, 512:576])
    md_ref[...] = c[:, 384:434]
    mi_ref[...] = c[:, 434:484]
    beff = lax.dot_general(bi_ref[...], wl0, (((1,), (1,)), ((), ())),
                           preferred_element_type=jnp.float32)
    pe_ref[...] = pos_ref[...] + beff + bl_ref[...]


def _prep(w_item, w_lin, b_item, b_lin, pos_table, small_tables):
    return pl.pallas_call(
        _prep_body,
        out_shape=(
            jax.ShapeDtypeStruct((H, NSC * H), jnp.float32),
            jax.ShapeDtypeStruct((OH_TOT, H), jnp.float32),
            jax.ShapeDtypeStruct((H, 50), jnp.float32),
            jax.ShapeDtypeStruct((H, 50), jnp.float32),
            jax.ShapeDtypeStruct((L, H), jnp.float32),
        ),
    )(w_item, w_lin, b_item.reshape(1, H), b_lin.reshape(1, H), pos_table,
      *small_tables)


def _main_body(*refs):
    gs = refs[:NSC]
    (sidx_ref, desc_ref, img_ref, m_ref, p_ref, md_ref, mi_ref,
     pe_ref, g_ref, b_ref, o_ref) = refs[NSC:]
    bf = jnp.bfloat16
    xg = jnp.concatenate([gs[f][0] for f in range(NSC)], axis=-1)
    y = lax.dot_general(xg.astype(bf), m_ref[...].astype(bf),
                        (((1,), (1,)), ((), ())),
                        preferred_element_type=jnp.float32)
    oh_parts = []
    for f in range(NOH):
        iota = lax.broadcasted_iota(jnp.int32, (OH_PAD[f], BR), 0)
        oh_parts.append((iota == sidx_ref[f, 0]).astype(bf))
    oht = jnp.concatenate(oh_parts, axis=0)
    y += lax.dot_general(oht, p_ref[...].astype(bf),
                         (((0,), (0,)), ((), ())),
                         preferred_element_type=jnp.float32)
    y += lax.dot_general(desc_ref[...].astype(bf), md_ref[...].astype(bf),
                         (((1,), (1,)), ((), ())),
                         preferred_element_type=jnp.float32)
    y += lax.dot_general(img_ref[...].astype(bf), mi_ref[...].astype(bf),
                         (((1,), (1,)), ((), ())),
                         preferred_element_type=jnp.float32)
    y = (y.reshape(BR // L, L, H) + pe_ref[...][None]).reshape(BR, H)
    mean = jnp.mean(y, axis=-1, keepdims=True)
    yc = y - mean
    var = jnp.mean(yc * yc, axis=-1, keepdims=True)
    o_ref[...] = yc * lax.rsqrt(var + 1e-12) * g_ref[...] + b_ref[...]


def _field_spec(f):
    return pl.BlockSpec((1, BR, H), lambda i, f=f: (f, i, 0))


def _main(gx, sidx, desc, img, m3, p, md, mi, pe, gamma, beta):
    return pl.pallas_call(
        _main_body,
        grid=(NBLK,),
        in_specs=[_field_spec(f) for f in range(NSC)] + [
            pl.BlockSpec((NOH, 1, 1, BR), lambda i: (0, i, 0, 0)),
            pl.BlockSpec((BR, 50), lambda i: (i, 0)),
            pl.BlockSpec((BR, 50), lambda i: (i, 0)),
            pl.BlockSpec((H, NSC * H), lambda i: (0, 0)),
            pl.BlockSpec((OH_TOT, H), lambda i: (0, 0)),
            pl.BlockSpec((H, 50), lambda i: (0, 0)),
            pl.BlockSpec((H, 50), lambda i: (0, 0)),
            pl.BlockSpec((L, H), lambda i: (0, 0)),
            pl.BlockSpec((1, H), lambda i: (0, 0)),
            pl.BlockSpec((1, H), lambda i: (0, 0)),
        ],
        out_specs=pl.BlockSpec((BR, H), lambda i: (i, 0)),
        out_shape=jax.ShapeDtypeStruct((ROWS, H), jnp.float32),
        compiler_params=pltpu.CompilerParams(
            dimension_semantics=("parallel",)),
    )(*([gx] * NSC), sidx, desc, img, m3, p, md, mi, pe, gamma, beta)


def kernel(input_ids, elapsed_time, event_type, product_action, hashed_url,
           price_bucket, number_of_category_hash, category_hash_first_level,
           category_hash_second_level, category_hash_third_level,
           description_vector, image_vector, hour, weekday, weekend,
           id_table, elapsed_table, event_table, pa_table, url_table,
           price_table, nch_table, c1_table, c2_table, c3_table, hour_table,
           weekday_table, weekend_table, pos_table, W_item, b_item, W_lin,
           b_lin, gamma, beta):
    pad = lambda t: jnp.pad(t, ((0, 0), (0, H - E)))
    sc_idx = [input_ids.reshape(ROWS),
              category_hash_first_level.reshape(ROWS),
              category_hash_second_level.reshape(ROWS),
              category_hash_third_level.reshape(ROWS),
              hashed_url.reshape(ROWS)]
    gx = _sc_gather(
        *sc_idx, pad(id_table), pad(c1_table), pad(c2_table),
        pad(c3_table), pad(url_table))
    sidx = jnp.stack([
        price_bucket.reshape(ROWS), number_of_category_hash.reshape(ROWS),
        elapsed_time.reshape(ROWS), event_type.reshape(ROWS),
        product_action.reshape(ROWS), hour.reshape(ROWS),
        weekday.reshape(ROWS), weekend.reshape(ROWS)])
    sidx = sidx.reshape(NOH, NBLK, 1, BR)
    m3, p, md, mi, pe = _prep(
        W_item, W_lin, b_item, b_lin, pos_table,
        (price_table, nch_table, elapsed_table, event_table, pa_table,
         hour_table, weekday_table, weekend_table))
    out = _main(gx, sidx, description_vector.reshape(ROWS, 50),
                image_vector.reshape(ROWS, 50), m3, p, md, mi, pe,
                gamma.reshape(1, H), beta.reshape(1, H))
    return out.reshape(B, L, H)
